# baseline (speedup 1.0000x reference)
import functools

import jax
import jax.numpy as jnp
from jax import lax
from jax.experimental import pallas as pl
from jax.experimental.pallas import tpu as pltpu
from jax.experimental.pallas import tpu_sc as plsc

_VOCAB = 1000000
_D = 64
_B = 4096
_H = 200
_NW = 32
_BHI = _B // 128
_NBLK = _H * _BHI
_PER_W = _NBLK // _NW


def _sc_gather_t(table, idx3):
    mesh = plsc.VectorSubcoreMesh(core_axis_name="c", subcore_axis_name="s")

    @functools.partial(
        pl.kernel,
        mesh=mesh,
        compiler_params=pltpu.CompilerParams(
            use_tc_tiling_on_sc=False, needs_layout_passes=False
        ),
        out_type=jax.ShapeDtypeStruct((_H, _D // 8, _BHI, 8, 128), jnp.float32),
        scratch_types=[
            pltpu.VMEM((_PER_W, 128), jnp.int32),
            pltpu.VMEM((2, 128, _D), jnp.float32),
            pltpu.VMEM((2, _D // 8, 8, 128), jnp.float32),
            pltpu.SemaphoreType.DMA,
            pltpu.SemaphoreType.DMA,
        ],
    )
    def k(table_hbm, idx_hbm, out_hbm, idx_v, stage, tblk, gsem, ssem):
        wid = lax.axis_index("s") * 2 + lax.axis_index("c")
        base = wid * _PER_W
        pltpu.sync_copy(idx_hbm.at[wid], idx_v)

        lane = lax.iota(jnp.int32, 16)

        def fire_g(j, buf):
            pltpu.async_copy(table_hbm.at[idx_v.at[j]], stage.at[buf], gsem)

        def wait_g(buf):
            pltpu.make_async_copy(
                table_hbm.at[pl.ds(0, 128)], stage.at[buf], gsem
            ).wait()

        def out_slice(j):
            bid = base + j
            return out_hbm.at[bid // _BHI, :, bid % _BHI]

        def transpose(buf):
            def gbody(g, carry):
                d1 = 16 * g + lane
                d0 = jnp.full((16,), buf, jnp.int32)
                for c in range(_D):
                    d2 = jnp.full((16,), c, jnp.int32)
                    col = plsc.load_gather(stage, [d0, d1, d2])
                    tblk[buf, c // 8, c % 8, pl.ds(16 * g, 16)] = col
                return carry

            lax.fori_loop(0, 8, gbody, 0)

        def fire_s(j, buf):
            pltpu.async_copy(tblk.at[buf], out_slice(j), ssem)

        def wait_s(j, buf):
            pltpu.make_async_copy(tblk.at[buf], out_slice(j), ssem).wait()

        def drain_one(p):
            pltpu.make_async_copy(tblk.at[p], out_hbm.at[0, :, 0], ssem).wait()

        fire_g(0, 0)
        for j in (0, 1):
            p = j % 2
            wait_g(p)
            fire_g(j + 1, 1 - p)
            transpose(p)
            fire_s(j, p)

        def body(j2, carry):
            for p in range(2):
                j = 2 * j2 + p
                wait_g(p)
                fire_g(j + 1, 1 - p)
                transpose(p)
                drain_one(p)
                fire_s(j, p)
            return carry

        lax.fori_loop(1, (_PER_W - 2) // 2, body, 0)

        for j in (_PER_W - 2, _PER_W - 1):
            p = j % 2
            wait_g(p)
            if j < _PER_W - 1:
                fire_g(j + 1, 1 - p)
            transpose(p)
            drain_one(p)
            fire_s(j, p)
        drain_one(0)
        drain_one(1)

    return k(table, idx3)


def _tc_mask(word_idx):
    def mk(idx_ref, o_ref):
        o_ref[...] = idx_ref[...] > 0

    return pl.pallas_call(
        mk,
        out_shape=jax.ShapeDtypeStruct((_B, _H), jnp.bool_),
    )(word_idx)


def kernel(word_idx, emb_table):
    idx3 = word_idx.transpose(1, 0).reshape(_NW, _PER_W, 128)
    emb5 = _sc_gather_t(emb_table, idx3)
    emb = emb5.transpose(2, 4, 0, 1, 3).reshape(_B, _H, _D)
    mask = _tc_mask(word_idx)
    return emb, mask


# speedup vs baseline: 1.2513x; 1.2513x over previous
import functools

import jax
import jax.numpy as jnp
from jax import lax
from jax.experimental import pallas as pl
from jax.experimental.pallas import tpu as pltpu
from jax.experimental.pallas import tpu_sc as plsc

_VOCAB = 1000000
_D = 64
_B = 4096
_H = 200
_NW = 32
_BHI = _B // 128
_NBLK = _H * _BHI
_PER_W = _NBLK // _NW
_G = 4
_NGRP = _PER_W // _G
_GROWS = _G * 8 * 128


def _sc_gather_t(table, idx3):
    mesh = plsc.VectorSubcoreMesh(core_axis_name="c", subcore_axis_name="s")

    @functools.partial(
        pl.kernel,
        mesh=mesh,
        compiler_params=pltpu.CompilerParams(
            use_tc_tiling_on_sc=False, needs_layout_passes=False
        ),
        out_type=jax.ShapeDtypeStruct((_H, 8, _BHI, 8, 128), jnp.float32),
        scratch_types=[
            pltpu.VMEM((_PER_W, 128), jnp.int32),
            pltpu.VMEM((8, 128, _D), jnp.float32),
            pltpu.VMEM((8, _G, 8, 128), jnp.float32),
            pltpu.SemaphoreType.DMA,
            pltpu.SemaphoreType.DMA,
        ],
    )
    def k(table_hbm, idx_hbm, out_hbm, idx_v, stage, tbig, gsem, ssem):
        wid = lax.axis_index("s") * 2 + lax.axis_index("c")
        base = wid * _PER_W
        pltpu.sync_copy(idx_hbm.at[wid], idx_v)

        lane = lax.iota(jnp.int32, 16)

        def fire_group(i):
            sb = (i % 2) * _G
            for q in range(_G):
                pltpu.async_copy(
                    table_hbm.at[idx_v.at[_G * i + q]], stage.at[sb + q], gsem
                )

        def wait_group():
            for q in range(_G):
                pltpu.make_async_copy(
                    table_hbm.at[pl.ds(0, 128)], stage.at[q], gsem
                ).wait()

        def transpose_group(i):
            sb = (i % 2) * _G

            @plsc.parallel_loop(0, _G * 8, unroll=2)
            def _(i2):
                q = i2 // 8
                g = i2 % 8
                d0 = jnp.full((16,), sb + q, jnp.int32)
                d1 = 16 * g + lane
                for c in range(_D):
                    d2 = jnp.full((16,), c, jnp.int32)
                    col = plsc.load_gather(stage, [d0, d1, d2])
                    tbig[c // 8, q, c % 8, pl.ds(16 * g, 16)] = col

        def fire_writes(i):
            bid0 = base + _G * i
            h = bid0 // _BHI
            boff = bid0 % _BHI
            for chi in range(8):
                pltpu.async_copy(
                    tbig.at[chi], out_hbm.at[h, chi, pl.ds(boff, _G)], ssem
                )

        def drain_writes():
            for chi in range(8):
                pltpu.make_async_copy(
                    tbig.at[chi], out_hbm.at[0, 0, pl.ds(0, _G)], ssem
                ).wait()

        fire_group(0)
        fire_group(1)

        wait_group()
        transpose_group(0)
        fire_group(2)
        fire_writes(0)

        def body(i, carry):
            wait_group()
            drain_writes()
            transpose_group(i)
            fire_group(i + 2)
            fire_writes(i)
            return carry

        lax.fori_loop(1, _NGRP - 2, body, 0)

        for i in (_NGRP - 2, _NGRP - 1):
            wait_group()
            drain_writes()
            transpose_group(i)
            fire_writes(i)
        drain_writes()

    return k(table, idx3)


def _tc_mask(word_idx):
    def mk(idx_ref, o_ref):
        o_ref[...] = idx_ref[...] > 0

    return pl.pallas_call(
        mk,
        out_shape=jax.ShapeDtypeStruct((_B, _H), jnp.bool_),
    )(word_idx)


def kernel(word_idx, emb_table):
    idx3 = word_idx.transpose(1, 0).reshape(_NW, _PER_W, 128)
    emb5 = _sc_gather_t(emb_table, idx3)
    emb = emb5.transpose(2, 4, 0, 1, 3).reshape(_B, _H, _D)
    mask = _tc_mask(word_idx)
    return emb, mask


# speedup vs baseline: 1.3856x; 1.1073x over previous
import functools

import jax
import jax.numpy as jnp
from jax import lax
from jax.experimental import pallas as pl
from jax.experimental.pallas import tpu as pltpu
from jax.experimental.pallas import tpu_sc as plsc

_VOCAB = 1000000
_D = 64
_B = 4096
_H = 200
_NW = 32
_BHI = _B // 128
_NBLK = _H * _BHI
_PER_W = _NBLK // _NW
_G = 4
_NGRP = _PER_W // _G
_GROWS = _G * 8 * 128


def _sc_gather_t(table, idx3):
    mesh = plsc.VectorSubcoreMesh(core_axis_name="c", subcore_axis_name="s")

    @functools.partial(
        pl.kernel,
        mesh=mesh,
        compiler_params=pltpu.CompilerParams(
            use_tc_tiling_on_sc=False, needs_layout_passes=False
        ),
        out_type=jax.ShapeDtypeStruct((_H, 8, _BHI, 8, 128), jnp.float32),
        scratch_types=[
            pltpu.VMEM((_PER_W, 128), jnp.int32),
            pltpu.VMEM((8, 128, _D), jnp.float32),
            pltpu.VMEM((8, _G, 8, 128), jnp.float32),
            pltpu.SemaphoreType.DMA,
            pltpu.SemaphoreType.DMA,
        ],
    )
    def k(table_hbm, idx_hbm, out_hbm, idx_v, stage, tbig, gsem, ssem):
        wid = lax.axis_index("s") * 2 + lax.axis_index("c")
        base = wid * _PER_W
        pltpu.sync_copy(idx_hbm.at[wid], idx_v)

        lane = lax.iota(jnp.int32, 16)

        def fire_group(i):
            sb = (i % 2) * _G
            for q in range(_G):
                pltpu.async_copy(
                    table_hbm.at[idx_v.at[_G * i + q]], stage.at[sb + q], gsem
                )

        def wait_group():
            for q in range(_G):
                pltpu.make_async_copy(
                    table_hbm.at[pl.ds(0, 128)], stage.at[q], gsem
                ).wait()

        def transpose_group(i):
            sb = (i % 2) * _G

            @plsc.parallel_loop(0, _G * 8 * _D, unroll=8)
            def _(i3):
                q = i3 // (8 * _D)
                g = (i3 // _D) % 8
                c = i3 % _D
                d0 = jnp.full((16,), sb + q, jnp.int32)
                d1 = 16 * g + lane
                d2 = jnp.full((16,), c, jnp.int32)
                col = plsc.load_gather(stage, [d0, d1, d2])
                tbig[c // 8, q, c % 8, pl.ds(16 * g, 16)] = col

        def fire_writes(i):
            bid0 = base + _G * i
            h = bid0 // _BHI
            boff = bid0 % _BHI
            for chi in range(8):
                pltpu.async_copy(
                    tbig.at[chi], out_hbm.at[h, chi, pl.ds(boff, _G)], ssem
                )

        def drain_writes():
            for chi in range(8):
                pltpu.make_async_copy(
                    tbig.at[chi], out_hbm.at[0, 0, pl.ds(0, _G)], ssem
                ).wait()

        fire_group(0)
        fire_group(1)

        wait_group()
        transpose_group(0)
        fire_group(2)
        fire_writes(0)

        def body(i, carry):
            wait_group()
            drain_writes()
            transpose_group(i)
            fire_group(i + 2)
            fire_writes(i)
            return carry

        lax.fori_loop(1, _NGRP - 2, body, 0)

        for i in (_NGRP - 2, _NGRP - 1):
            wait_group()
            drain_writes()
            transpose_group(i)
            fire_writes(i)
        drain_writes()

    return k(table, idx3)


def _tc_mask(word_idx):
    def mk(idx_ref, o_ref):
        o_ref[...] = idx_ref[...] > 0

    return pl.pallas_call(
        mk,
        out_shape=jax.ShapeDtypeStruct((_B, _H), jnp.bool_),
    )(word_idx)


def kernel(word_idx, emb_table):
    idx3 = word_idx.transpose(1, 0).reshape(_NW, _PER_W, 128)
    emb5 = _sc_gather_t(emb_table, idx3)
    emb = emb5.transpose(2, 4, 0, 1, 3).reshape(_B, _H, _D)
    mask = _tc_mask(word_idx)
    return emb, mask


# speedup vs baseline: 2.2518x; 1.6251x over previous
import functools

import jax
import jax.numpy as jnp
from jax import lax
from jax.experimental import pallas as pl
from jax.experimental.pallas import tpu as pltpu
from jax.experimental.pallas import tpu_sc as plsc

_VOCAB = 1000000
_D = 64
_B = 4096
_H = 200
_NW = 32
_BHI = _B // 128
_NBLK = _H * _BHI
_PER_W = _NBLK // _NW
_G = 4
_NGRP = _PER_W // _G


def _sc_gather_t(table, idx3):
    mesh = plsc.VectorSubcoreMesh(core_axis_name="c", subcore_axis_name="s")

    @functools.partial(
        pl.kernel,
        mesh=mesh,
        compiler_params=pltpu.CompilerParams(
            use_tc_tiling_on_sc=False, needs_layout_passes=False
        ),
        out_type=jax.ShapeDtypeStruct((_H, 8, _BHI, 8, 128), jnp.float32),
        scratch_types=[
            pltpu.VMEM((_PER_W, 128), jnp.int32),
            pltpu.VMEM((8, 128, _D), jnp.float32),
            pltpu.VMEM((64 * 65,), jnp.float32),
            pltpu.VMEM((8, _G, 8, 128), jnp.float32),
            pltpu.SemaphoreType.DMA,
            pltpu.SemaphoreType.DMA,
        ],
    )
    def k(table_hbm, idx_hbm, out_hbm, idx_v, stage, skew, tbig, gsem, ssem):
        wid = lax.axis_index("s") * 2 + lax.axis_index("c")
        base = wid * _PER_W
        pltpu.sync_copy(idx_hbm.at[wid], idx_v)

        lane = lax.iota(jnp.int32, 16)
        g65 = [(16 * gg + lane) * 65 for gg in range(4)]

        def fire_group(i):
            sb = (i % 2) * _G
            for q in range(_G):
                pltpu.async_copy(
                    table_hbm.at[idx_v.at[_G * i + q]], stage.at[sb + q], gsem
                )

        def wait_group():
            for q in range(_G):
                pltpu.make_async_copy(
                    table_hbm.at[pl.ds(0, 128)], stage.at[q], gsem
                ).wait()

        def transpose_chunk(s, q):
            for half in range(2):
                b0 = 64 * half

                @plsc.parallel_loop(0, 64, unroll=4)
                def _(b):
                    boff = b * 65
                    for c0 in range(0, _D, 16):
                        skew[pl.ds(boff + c0, 16)] = stage[
                            s, b0 + b, pl.ds(c0, 16)
                        ]

                @plsc.parallel_loop(0, _D, unroll=4)
                def _(c):
                    for gg in range(4):
                        col = plsc.load_gather(skew, [g65[gg] + c])
                        tbig[c >> 3, q, c & 7, pl.ds(b0 + 16 * gg, 16)] = col

        def transpose_group(i):
            sb = (i % 2) * _G
            for q in range(_G):
                transpose_chunk(sb + q, q)

        def fire_writes(i):
            bid0 = base + _G * i
            h = bid0 // _BHI
            boff = bid0 % _BHI
            for chi in range(8):
                pltpu.async_copy(
                    tbig.at[chi], out_hbm.at[h, chi, pl.ds(boff, _G)], ssem
                )

        def drain_writes():
            for chi in range(8):
                pltpu.make_async_copy(
                    tbig.at[chi], out_hbm.at[0, 0, pl.ds(0, _G)], ssem
                ).wait()

        fire_group(0)
        fire_group(1)

        wait_group()
        transpose_group(0)
        fire_group(2)
        fire_writes(0)

        def body(i, carry):
            wait_group()
            drain_writes()
            transpose_group(i)
            fire_group(i + 2)
            fire_writes(i)
            return carry

        lax.fori_loop(1, _NGRP - 2, body, 0)

        for i in (_NGRP - 2, _NGRP - 1):
            wait_group()
            drain_writes()
            transpose_group(i)
            fire_writes(i)
        drain_writes()

    return k(table, idx3)


def _tc_mask(word_idx):
    def mk(idx_ref, o_ref):
        o_ref[...] = idx_ref[...] > 0

    return pl.pallas_call(
        mk,
        out_shape=jax.ShapeDtypeStruct((_B, _H), jnp.bool_),
    )(word_idx)


def kernel(word_idx, emb_table):
    idx3 = word_idx.transpose(1, 0).reshape(_NW, _PER_W, 128)
    emb5 = _sc_gather_t(emb_table, idx3)
    emb = emb5.transpose(2, 4, 0, 1, 3).reshape(_B, _H, _D)
    mask = _tc_mask(word_idx)
    return emb, mask


# speedup vs baseline: 2.2623x; 1.0047x over previous
import functools

import jax
import jax.numpy as jnp
from jax import lax
from jax.experimental import pallas as pl
from jax.experimental.pallas import tpu as pltpu
from jax.experimental.pallas import tpu_sc as plsc

_VOCAB = 1000000
_D = 64
_B = 4096
_H = 200
_NW = 32
_BHI = _B // 128
_NBLK = _H * _BHI
_PER_W = _NBLK // _NW
_G = 4
_NGRP = _PER_W // _G


def _sc_gather_t(table, idx3):
    mesh = plsc.VectorSubcoreMesh(core_axis_name="c", subcore_axis_name="s")

    @functools.partial(
        pl.kernel,
        mesh=mesh,
        compiler_params=pltpu.CompilerParams(
            use_tc_tiling_on_sc=False, needs_layout_passes=False
        ),
        out_type=jax.ShapeDtypeStruct((_H, 8, _BHI, 8, 128), jnp.float32),
        scratch_types=[
            pltpu.VMEM((_PER_W, 128), jnp.int32),
            pltpu.VMEM((8, 128, _D), jnp.float32),
            pltpu.VMEM((64 * 65,), jnp.float32),
            pltpu.VMEM((8, _G, 8, 128), jnp.float32),
            pltpu.SemaphoreType.DMA,
            pltpu.SemaphoreType.DMA,
        ],
    )
    def k(table_hbm, idx_hbm, out_hbm, idx_v, stage, skew, tbig, gsem, ssem):
        wid = lax.axis_index("s") * 2 + lax.axis_index("c")
        base = wid * _PER_W
        pltpu.sync_copy(idx_hbm.at[wid], idx_v)

        lane = lax.iota(jnp.int32, 16)
        g65 = [(16 * gg + lane) * 65 for gg in range(4)]

        def fire_group(i):
            sb = (i % 2) * _G
            for q in range(_G):
                pltpu.async_copy(
                    table_hbm.at[idx_v.at[_G * i + q]], stage.at[sb + q], gsem
                )

        def wait_group():
            for q in range(_G):
                pltpu.make_async_copy(
                    table_hbm.at[pl.ds(0, 128)], stage.at[q], gsem
                ).wait()

        def transpose_chunk(s, q):
            for half in range(2):
                b0 = 64 * half

                @plsc.parallel_loop(0, 64, unroll=8)
                def _(b):
                    boff = b * 65
                    for c0 in range(0, _D, 16):
                        skew[pl.ds(boff + c0, 16)] = stage[
                            s, b0 + b, pl.ds(c0, 16)
                        ]

                @plsc.parallel_loop(0, _D, unroll=8)
                def _(c):
                    for gg in range(4):
                        col = plsc.load_gather(skew, [g65[gg] + c])
                        tbig[c >> 3, q, c & 7, pl.ds(b0 + 16 * gg, 16)] = col

        def transpose_group(i):
            sb = (i % 2) * _G
            for q in range(_G):
                transpose_chunk(sb + q, q)

        def fire_writes(i):
            bid0 = base + _G * i
            h = bid0 // _BHI
            boff = bid0 % _BHI
            for chi in range(8):
                pltpu.async_copy(
                    tbig.at[chi], out_hbm.at[h, chi, pl.ds(boff, _G)], ssem
                )

        def drain_writes():
            for chi in range(8):
                pltpu.make_async_copy(
                    tbig.at[chi], out_hbm.at[0, 0, pl.ds(0, _G)], ssem
                ).wait()

        fire_group(0)
        fire_group(1)

        wait_group()
        transpose_group(0)
        fire_group(2)
        fire_writes(0)

        def body(i, carry):
            wait_group()
            drain_writes()
            transpose_group(i)
            fire_group(i + 2)
            fire_writes(i)
            return carry

        lax.fori_loop(1, _NGRP - 2, body, 0)

        for i in (_NGRP - 2, _NGRP - 1):
            wait_group()
            drain_writes()
            transpose_group(i)
            fire_writes(i)
        drain_writes()

    return k(table, idx3)


def _tc_mask(word_idx):
    def mk(idx_ref, o_ref):
        o_ref[...] = idx_ref[...] > 0

    return pl.pallas_call(
        mk,
        out_shape=jax.ShapeDtypeStruct((_B, _H), jnp.bool_),
    )(word_idx)


def kernel(word_idx, emb_table):
    idx3 = word_idx.transpose(1, 0).reshape(_NW, _PER_W, 128)
    emb5 = _sc_gather_t(emb_table, idx3)
    emb = emb5.transpose(2, 4, 0, 1, 3).reshape(_B, _H, _D)
    mask = _tc_mask(word_idx)
    return emb, mask
